# Initial kernel scaffold; baseline (speedup 1.0000x reference)
import jax
import jax.numpy as jnp
from jax.experimental import pallas as pl


def kernel(x_user, x_item, edge_user_item, edge_item_user, p_user_item, p_item_user, g_user, b_user, g_item, b_item):
    raise NotImplementedError("write your pallas kernel here")


# speedup vs baseline: 3.8203x; 3.8203x over previous
import functools

import jax
import jax.numpy as jnp
from jax import lax
from jax.experimental import pallas as pl
from jax.experimental.pallas import tpu as pltpu
from jax.experimental.pallas import tpu_sc as plsc

N = 10000
D = 256
E = 160000
DH = 128
C = 128
L = 16
NSUB = 16
ROWS_PER_TILE = N // NSUB
FLUSH_ROWS = (N // NSUB) // 8 * 8
CHUNKS = E // C
CHUNKS_PER_TILE = -(-CHUNKS // NSUB)
ALPHA = 0.5
EPS = 1e-5


def _sc_aggregate(xu_cat, xi_cat, src_ui, dst_ui, src_iu, dst_iu):
    mesh = plsc.VectorSubcoreMesh(core_axis_name="core",
                                  subcore_axis_name="subcore")
    out_type = [
        jax.ShapeDtypeStruct((2 * N, DH), jnp.float32),
        jax.ShapeDtypeStruct((2 * N, DH), jnp.float32),
        jax.ShapeDtypeStruct((N,), jnp.float32),
        jax.ShapeDtypeStruct((N,), jnp.float32),
    ]
    scratch = [
        pltpu.VMEM_SHARED((N, DH), jnp.float32),
        pltpu.VMEM_SHARED((N,), jnp.float32),
        pltpu.VMEM((C,), jnp.int32),
        pltpu.VMEM((C,), jnp.int32),
        pltpu.VMEM((C, DH), jnp.float32),
        pltpu.VMEM((C,), jnp.float32),
        pltpu.VMEM((C,), jnp.float32),
        pltpu.VMEM((FLUSH_ROWS + 16,), jnp.float32),
        pltpu.SemaphoreType.DMA,
    ]

    @functools.partial(pl.kernel, mesh=mesh, out_type=out_type,
                       scratch_types=scratch)
    def k(xu_hbm, xi_hbm, sui_hbm, dui_hbm, siu_hbm, diu_hbm,
          aggi_hbm, aggu_hbm, degi_hbm, degu_hbm,
          sh_agg, sh_deg, idx_v, dst_v, rows_v, ones_v, zdeg_v, dbuf, sem):
        c = lax.axis_index("core")
        s = lax.axis_index("subcore")
        coff = c * N

        for j in range(C // L):
            ones_v[pl.ds(j * L, L)] = jnp.full((L,), 1.0, jnp.float32)
            zdeg_v[pl.ds(j * L, L)] = jnp.zeros((L,), jnp.float32)

        for rel in range(2):
            x_hbm = (xu_hbm, xi_hbm)[rel]
            s_hbm = (sui_hbm, siu_hbm)[rel]
            d_hbm = (dui_hbm, diu_hbm)[rel]
            agg_hbm = (aggi_hbm, aggu_hbm)[rel]
            deg_hbm = (degi_hbm, degu_hbm)[rel]

            @pl.loop(0, C)
            def _(i):
                for j in range(DH // L):
                    rows_v[i, pl.ds(j * L, L)] = jnp.zeros((L,), jnp.float32)

            fb = s * FLUSH_ROWS
            nfull = FLUSH_ROWS // C
            for q in range(nfull):
                pltpu.sync_copy(rows_v, sh_agg.at[pl.ds(fb + q * C, C)])
                pltpu.sync_copy(zdeg_v, sh_deg.at[pl.ds(fb + q * C, C)])
            rem = FLUSH_ROWS % C
            if rem:
                pltpu.sync_copy(rows_v.at[pl.ds(0, rem)],
                                sh_agg.at[pl.ds(fb + nfull * C, rem)])
                pltpu.sync_copy(zdeg_v.at[pl.ds(0, rem)],
                                sh_deg.at[pl.ds(fb + nfull * C, rem)])

            @pl.when(s == NSUB - 1)
            def _():
                tb = NSUB * FLUSH_ROWS
                pltpu.sync_copy(rows_v.at[pl.ds(0, N - tb)],
                                sh_agg.at[pl.ds(tb, N - tb)])
                pltpu.sync_copy(zdeg_v.at[pl.ds(0, N - tb)],
                                sh_deg.at[pl.ds(tb, N - tb)])
            plsc.subcore_barrier()

            @pl.loop(0, CHUNKS_PER_TILE)
            def _(kk):
                q = s + kk * NSUB

                @pl.when(q < CHUNKS)
                def _():
                    base = pl.multiple_of(q * C, C)
                    pltpu.sync_copy(s_hbm.at[pl.ds(base, C)], idx_v)
                    pltpu.sync_copy(d_hbm.at[pl.ds(base, C)], dst_v)
                    for j in range(C // L):
                        sl = pl.ds(j * L, L)
                        idx_v[sl] = idx_v[sl] + coff
                    pltpu.async_copy(x_hbm.at[idx_v], rows_v, sem).wait()
                    pltpu.sync_copy(rows_v, sh_agg.at[dst_v], add=True)

                    @pl.when(c == rel)
                    def _():
                        pltpu.sync_copy(ones_v, sh_deg.at[dst_v], add=True)

            plsc.subcore_barrier()

            lens = [C] * (FLUSH_ROWS // C) + [FLUSH_ROWS % C]
            off = 0
            for ln in lens:
                pltpu.sync_copy(sh_agg.at[pl.ds(fb + off, ln)],
                                rows_v.at[pl.ds(0, ln)])
                pltpu.sync_copy(rows_v.at[pl.ds(0, ln)],
                                agg_hbm.at[pl.ds(coff + fb + off, ln)])
                off += ln

            @pl.when(s == NSUB - 1)
            def _():
                tb = NSUB * FLUSH_ROWS
                pltpu.sync_copy(sh_agg.at[pl.ds(tb, N - tb)],
                                rows_v.at[pl.ds(0, N - tb)])
                pltpu.sync_copy(rows_v.at[pl.ds(0, N - tb)],
                                agg_hbm.at[pl.ds(coff + tb, N - tb)])

            @pl.when(c == rel)
            def _():
                pltpu.sync_copy(sh_deg.at[pl.ds(fb, FLUSH_ROWS)],
                                dbuf.at[pl.ds(0, FLUSH_ROWS)])
                pltpu.sync_copy(dbuf.at[pl.ds(0, FLUSH_ROWS)],
                                deg_hbm.at[pl.ds(fb, FLUSH_ROWS)])

                @pl.when(s == NSUB - 1)
                def _():
                    tb = NSUB * FLUSH_ROWS
                    pltpu.sync_copy(sh_deg.at[pl.ds(tb, N - tb)],
                                    dbuf.at[pl.ds(0, N - tb)])
                    pltpu.sync_copy(dbuf.at[pl.ds(0, N - tb)],
                                    deg_hbm.at[pl.ds(tb, N - tb)])

            plsc.subcore_barrier()

    return k(xu_cat, xi_cat, src_ui, dst_ui, src_iu, dst_iu)


def _norm_body(xu, au, du, xi, ai, di, pu, pi, gu, bu, gi, bi, ou, oi):
    for x, a, d, p, g, b, o in ((xu, au, du, pu, gu, bu, ou),
                                (xi, ai, di, pi, gi, bi, oi)):
        h = x[...] + ALPHA * (a[...] * p[...]) / jnp.maximum(d[...], 1.0)
        mu = jnp.mean(h, axis=-1, keepdims=True)
        var = jnp.mean((h - mu) ** 2, axis=-1, keepdims=True)
        o[...] = (h - mu) * lax.rsqrt(var + EPS) * g[...] + b[...]


def _tc_normalize(x_user, agg_user, deg_user, x_item, agg_item, deg_item,
                  p_iu, p_ui, g_u, b_u, g_i, b_i):
    R = 1000
    bs_feat = pl.BlockSpec((R, D), lambda i: (i, 0))
    bs_deg = pl.BlockSpec((R, 1), lambda i: (i, 0))
    bs_vec = pl.BlockSpec((1, D), lambda i: (0, 0))
    return pl.pallas_call(
        _norm_body,
        grid=(N // R,),
        in_specs=[bs_feat, bs_feat, bs_deg, bs_feat, bs_feat, bs_deg,
                  bs_vec, bs_vec, bs_vec, bs_vec, bs_vec, bs_vec],
        out_specs=[bs_feat, bs_feat],
        out_shape=[jax.ShapeDtypeStruct((N, D), jnp.float32),
                   jax.ShapeDtypeStruct((N, D), jnp.float32)],
    )(x_user, agg_user, deg_user, x_item, agg_item, deg_item,
      p_iu.reshape(1, D), p_ui.reshape(1, D),
      g_u.reshape(1, D), b_u.reshape(1, D),
      g_i.reshape(1, D), b_i.reshape(1, D))


def kernel(x_user, x_item, edge_user_item, edge_item_user,
           p_user_item, p_item_user, g_user, b_user, g_item, b_item):
    xu_cat = jnp.concatenate([x_user[:, :DH], x_user[:, DH:]], axis=0)
    xi_cat = jnp.concatenate([x_item[:, :DH], x_item[:, DH:]], axis=0)
    aggi_cat, aggu_cat, deg_item, deg_user = _sc_aggregate(
        xu_cat, xi_cat,
        edge_user_item[0], edge_user_item[1],
        edge_item_user[0], edge_item_user[1])
    agg_item = jnp.concatenate([aggi_cat[:N], aggi_cat[N:]], axis=1)
    agg_user = jnp.concatenate([aggu_cat[:N], aggu_cat[N:]], axis=1)
    out_user, out_item = _tc_normalize(
        x_user, agg_user, deg_user.reshape(N, 1),
        x_item, agg_item, deg_item.reshape(N, 1),
        p_item_user, p_user_item, g_user, b_user, g_item, b_item)
    return (out_user, out_item)


# speedup vs baseline: 5.6708x; 1.4844x over previous
import functools

import jax
import jax.numpy as jnp
from jax import lax
from jax.experimental import pallas as pl
from jax.experimental.pallas import tpu as pltpu
from jax.experimental.pallas import tpu_sc as plsc

N = 10000
D = 256
E = 160000
DH = 128
C = 128
L = 16
NSUB = 16
ROWS_PER_TILE = N // NSUB
FLUSH_ROWS = (N // NSUB) // 8 * 8
CHUNKS = E // C
CHUNKS_PER_TILE = -(-CHUNKS // NSUB)
ALPHA = 0.5
EPS = 1e-5


def _sc_aggregate(xu_cat, xi_cat, src_ui, dst_ui, src_iu, dst_iu):
    mesh = plsc.VectorSubcoreMesh(core_axis_name="core",
                                  subcore_axis_name="subcore")
    out_type = [
        jax.ShapeDtypeStruct((2 * N, DH), jnp.float32),
        jax.ShapeDtypeStruct((2 * N, DH), jnp.float32),
        jax.ShapeDtypeStruct((N,), jnp.float32),
        jax.ShapeDtypeStruct((N,), jnp.float32),
    ]
    scratch = [
        pltpu.VMEM_SHARED((N, DH), jnp.float32),
        pltpu.VMEM_SHARED((N,), jnp.float32),
        pltpu.VMEM((C,), jnp.int32),
        pltpu.VMEM((C,), jnp.int32),
        pltpu.VMEM((C, DH), jnp.float32),
        pltpu.VMEM((C,), jnp.int32),
        pltpu.VMEM((C,), jnp.int32),
        pltpu.VMEM((C, DH), jnp.float32),
        pltpu.VMEM((C,), jnp.float32),
        pltpu.VMEM((C,), jnp.float32),
        pltpu.VMEM((FLUSH_ROWS + 16,), jnp.float32),
        pltpu.SemaphoreType.DMA,
        pltpu.SemaphoreType.DMA,
    ]

    @functools.partial(pl.kernel, mesh=mesh, out_type=out_type,
                       scratch_types=scratch)
    def k(xu_hbm, xi_hbm, sui_hbm, dui_hbm, siu_hbm, diu_hbm,
          aggi_hbm, aggu_hbm, degi_hbm, degu_hbm,
          sh_agg, sh_deg, idx0, dst0, rows0, idx1, dst1, rows1,
          ones_v, zdeg_v, dbuf, sem0, sem1):
        c = lax.axis_index("core")
        s = lax.axis_index("subcore")
        coff = c * N

        for j in range(C // L):
            ones_v[pl.ds(j * L, L)] = jnp.full((L,), 1.0, jnp.float32)
            zdeg_v[pl.ds(j * L, L)] = jnp.zeros((L,), jnp.float32)

        for rel in range(2):
            x_hbm = (xu_hbm, xi_hbm)[rel]
            s_hbm = (sui_hbm, siu_hbm)[rel]
            d_hbm = (dui_hbm, diu_hbm)[rel]
            agg_hbm = (aggi_hbm, aggu_hbm)[rel]
            deg_hbm = (degi_hbm, degu_hbm)[rel]

            @pl.loop(0, C)
            def _(i):
                for j in range(DH // L):
                    rows0[i, pl.ds(j * L, L)] = jnp.zeros((L,), jnp.float32)

            fb = s * FLUSH_ROWS
            nfull = FLUSH_ROWS // C
            for q in range(nfull):
                pltpu.sync_copy(rows0, sh_agg.at[pl.ds(fb + q * C, C)])
                pltpu.sync_copy(zdeg_v, sh_deg.at[pl.ds(fb + q * C, C)])
            rem = FLUSH_ROWS % C
            if rem:
                pltpu.sync_copy(rows0.at[pl.ds(0, rem)],
                                sh_agg.at[pl.ds(fb + nfull * C, rem)])
                pltpu.sync_copy(zdeg_v.at[pl.ds(0, rem)],
                                sh_deg.at[pl.ds(fb + nfull * C, rem)])

            @pl.when(s == NSUB - 1)
            def _():
                tb = NSUB * FLUSH_ROWS
                pltpu.sync_copy(rows0.at[pl.ds(0, N - tb)],
                                sh_agg.at[pl.ds(tb, N - tb)])
                pltpu.sync_copy(zdeg_v.at[pl.ds(0, N - tb)],
                                sh_deg.at[pl.ds(tb, N - tb)])
            plsc.subcore_barrier()

            def do_idx(j, ib, db):
                q = s + j * NSUB

                @pl.when(q < CHUNKS)
                def _():
                    base = pl.multiple_of(q * C, C)
                    pltpu.sync_copy(s_hbm.at[pl.ds(base, C)], ib)
                    pltpu.sync_copy(d_hbm.at[pl.ds(base, C)], db)
                    for jj in range(C // L):
                        sl = pl.ds(jj * L, L)
                        ib[sl] = ib[sl] + coff

            def gather_start(j, ib, rb, sem):
                q = s + j * NSUB

                @pl.when(q < CHUNKS)
                def _():
                    pltpu.async_copy(x_hbm.at[ib], rb, sem)

            def gather_wait(j, ib, rb, sem):
                q = s + j * NSUB

                @pl.when(q < CHUNKS)
                def _():
                    pltpu.make_async_copy(x_hbm.at[ib], rb, sem).wait()

            def do_scatter(j, rb, db):
                q = s + j * NSUB

                @pl.when(q < CHUNKS)
                def _():
                    pltpu.sync_copy(rb, sh_agg.at[db], add=True)

                    @pl.when(c == rel)
                    def _():
                        pltpu.sync_copy(ones_v, sh_deg.at[db], add=True)

            do_idx(0, idx0, dst0)
            gather_start(0, idx0, rows0, sem0)

            @pl.loop(0, (CHUNKS_PER_TILE + 1) // 2)
            def _(t):
                ja = 2 * t
                jb = 2 * t + 1
                jn = 2 * t + 2
                do_idx(jb, idx1, dst1)
                gather_wait(ja, idx0, rows0, sem0)
                gather_start(jb, idx1, rows1, sem1)
                do_scatter(ja, rows0, dst0)
                do_idx(jn, idx0, dst0)
                gather_wait(jb, idx1, rows1, sem1)
                gather_start(jn, idx0, rows0, sem0)
                do_scatter(jb, rows1, dst1)

            plsc.subcore_barrier()

            lens = [C] * (FLUSH_ROWS // C) + [FLUSH_ROWS % C]
            off = 0
            for ln in lens:
                pltpu.sync_copy(sh_agg.at[pl.ds(fb + off, ln)],
                                rows0.at[pl.ds(0, ln)])
                pltpu.sync_copy(rows0.at[pl.ds(0, ln)],
                                agg_hbm.at[pl.ds(coff + fb + off, ln)])
                off += ln

            @pl.when(s == NSUB - 1)
            def _():
                tb = NSUB * FLUSH_ROWS
                pltpu.sync_copy(sh_agg.at[pl.ds(tb, N - tb)],
                                rows0.at[pl.ds(0, N - tb)])
                pltpu.sync_copy(rows0.at[pl.ds(0, N - tb)],
                                agg_hbm.at[pl.ds(coff + tb, N - tb)])

            @pl.when(c == rel)
            def _():
                pltpu.sync_copy(sh_deg.at[pl.ds(fb, FLUSH_ROWS)],
                                dbuf.at[pl.ds(0, FLUSH_ROWS)])
                pltpu.sync_copy(dbuf.at[pl.ds(0, FLUSH_ROWS)],
                                deg_hbm.at[pl.ds(fb, FLUSH_ROWS)])

                @pl.when(s == NSUB - 1)
                def _():
                    tb = NSUB * FLUSH_ROWS
                    pltpu.sync_copy(sh_deg.at[pl.ds(tb, N - tb)],
                                    dbuf.at[pl.ds(0, N - tb)])
                    pltpu.sync_copy(dbuf.at[pl.ds(0, N - tb)],
                                    deg_hbm.at[pl.ds(tb, N - tb)])

            plsc.subcore_barrier()

    return k(xu_cat, xi_cat, src_ui, dst_ui, src_iu, dst_iu)


def _norm_body(xu, au, du, xi, ai, di, pu, pi, gu, bu, gi, bi, ou, oi):
    for x, a, d, p, g, b, o in ((xu, au, du, pu, gu, bu, ou),
                                (xi, ai, di, pi, gi, bi, oi)):
        h = x[...] + ALPHA * (a[...] * p[...]) / jnp.maximum(d[...], 1.0)
        mu = jnp.mean(h, axis=-1, keepdims=True)
        var = jnp.mean((h - mu) ** 2, axis=-1, keepdims=True)
        o[...] = (h - mu) * lax.rsqrt(var + EPS) * g[...] + b[...]


def _tc_normalize(x_user, agg_user, deg_user, x_item, agg_item, deg_item,
                  p_iu, p_ui, g_u, b_u, g_i, b_i):
    R = 1000
    bs_feat = pl.BlockSpec((R, D), lambda i: (i, 0))
    bs_deg = pl.BlockSpec((R, 1), lambda i: (i, 0))
    bs_vec = pl.BlockSpec((1, D), lambda i: (0, 0))
    return pl.pallas_call(
        _norm_body,
        grid=(N // R,),
        in_specs=[bs_feat, bs_feat, bs_deg, bs_feat, bs_feat, bs_deg,
                  bs_vec, bs_vec, bs_vec, bs_vec, bs_vec, bs_vec],
        out_specs=[bs_feat, bs_feat],
        out_shape=[jax.ShapeDtypeStruct((N, D), jnp.float32),
                   jax.ShapeDtypeStruct((N, D), jnp.float32)],
    )(x_user, agg_user, deg_user, x_item, agg_item, deg_item,
      p_iu.reshape(1, D), p_ui.reshape(1, D),
      g_u.reshape(1, D), b_u.reshape(1, D),
      g_i.reshape(1, D), b_i.reshape(1, D))


def kernel(x_user, x_item, edge_user_item, edge_item_user,
           p_user_item, p_item_user, g_user, b_user, g_item, b_item):
    xu_cat = jnp.concatenate([x_user[:, :DH], x_user[:, DH:]], axis=0)
    xi_cat = jnp.concatenate([x_item[:, :DH], x_item[:, DH:]], axis=0)
    aggi_cat, aggu_cat, deg_item, deg_user = _sc_aggregate(
        xu_cat, xi_cat,
        edge_user_item[0], edge_user_item[1],
        edge_item_user[0], edge_item_user[1])
    agg_item = jnp.concatenate([aggi_cat[:N], aggi_cat[N:]], axis=1)
    agg_user = jnp.concatenate([aggu_cat[:N], aggu_cat[N:]], axis=1)
    out_user, out_item = _tc_normalize(
        x_user, agg_user, deg_user.reshape(N, 1),
        x_item, agg_item, deg_item.reshape(N, 1),
        p_item_user, p_user_item, g_user, b_user, g_item, b_item)
    return (out_user, out_item)


# speedup vs baseline: 5.7949x; 1.0219x over previous
import functools

import jax
import jax.numpy as jnp
from jax import lax
from jax.experimental import pallas as pl
from jax.experimental.pallas import tpu as pltpu
from jax.experimental.pallas import tpu_sc as plsc

N = 10000
D = 256
E = 160000
DH = 128
C = 80
L = 16
NSUB = 16
EPT = E // NSUB
CPT = EPT // C
SUPER = 25
SUPER_E = SUPER * C
FLUSH_ROWS = (N // NSUB) // 8 * 8
ALPHA = 0.5
EPS = 1e-5


def _sc_aggregate(xu_cat, xi_cat, src_ui, dst_ui, src_iu, dst_iu):
    mesh = plsc.VectorSubcoreMesh(core_axis_name="core",
                                  subcore_axis_name="subcore")
    out_type = [
        jax.ShapeDtypeStruct((2 * N, DH), jnp.float32),
        jax.ShapeDtypeStruct((2 * N, DH), jnp.float32),
        jax.ShapeDtypeStruct((N,), jnp.float32),
        jax.ShapeDtypeStruct((N,), jnp.float32),
    ]
    scratch = [
        pltpu.VMEM_SHARED((N, DH), jnp.float32),
        pltpu.VMEM_SHARED((N,), jnp.float32),
        pltpu.VMEM((SUPER_E,), jnp.int32),
        pltpu.VMEM((SUPER_E,), jnp.int32),
        pltpu.VMEM((C,), jnp.int32),
        pltpu.VMEM((C,), jnp.int32),
        pltpu.VMEM((C, DH), jnp.float32),
        pltpu.VMEM((C,), jnp.int32),
        pltpu.VMEM((C,), jnp.int32),
        pltpu.VMEM((C, DH), jnp.float32),
        pltpu.VMEM((C,), jnp.float32),
        pltpu.VMEM((C,), jnp.float32),
        pltpu.VMEM((FLUSH_ROWS + 16,), jnp.float32),
        pltpu.SemaphoreType.DMA,
        pltpu.SemaphoreType.DMA,
        pltpu.SemaphoreType.DMA,
        pltpu.SemaphoreType.DMA,
        pltpu.SemaphoreType.DMA,
        pltpu.SemaphoreType.DMA,
    ]

    @functools.partial(pl.kernel, mesh=mesh, out_type=out_type,
                       scratch_types=scratch)
    def k(xu_hbm, xi_hbm, sui_hbm, dui_hbm, siu_hbm, diu_hbm,
          aggi_hbm, aggu_hbm, degi_hbm, degu_hbm,
          sh_agg, sh_deg, srcbig, dstbig,
          idx0, dst0, rows0, idx1, dst1, rows1,
          ones_v, zdeg_v, dbuf,
          gsem0, gsem1, ssem0, ssem1, dsem0, dsem1):
        c = lax.axis_index("core")
        s = lax.axis_index("subcore")
        coff = c * N
        ebase = s * EPT

        for j in range(C // L):
            ones_v[pl.ds(j * L, L)] = jnp.full((L,), 1.0, jnp.float32)
            zdeg_v[pl.ds(j * L, L)] = jnp.zeros((L,), jnp.float32)

        B0 = (idx0, dst0, rows0, gsem0, ssem0, dsem0)
        B1 = (idx1, dst1, rows1, gsem1, ssem1, dsem1)

        for rel in range(2):
            x_hbm = (xu_hbm, xi_hbm)[rel]
            s_hbm = (sui_hbm, siu_hbm)[rel]
            d_hbm = (dui_hbm, diu_hbm)[rel]
            agg_hbm = (aggi_hbm, aggu_hbm)[rel]
            deg_hbm = (degi_hbm, degu_hbm)[rel]

            @pl.loop(0, C)
            def _(i):
                for j in range(DH // L):
                    rows0[i, pl.ds(j * L, L)] = jnp.zeros((L,), jnp.float32)

            fb = s * FLUSH_ROWS
            zlens = [C] * (FLUSH_ROWS // C) + [FLUSH_ROWS % C]
            off = 0
            for ln in zlens:
                pltpu.sync_copy(rows0.at[pl.ds(0, ln)],
                                sh_agg.at[pl.ds(fb + off, ln)])
                off += ln
            doff = 0
            for ln in zlens:
                pltpu.sync_copy(zdeg_v.at[pl.ds(0, ln)],
                                sh_deg.at[pl.ds(fb + doff, ln)])
                doff += ln

            @pl.when(s == NSUB - 1)
            def _():
                tb = NSUB * FLUSH_ROWS
                pltpu.sync_copy(rows0.at[pl.ds(0, N - tb)],
                                sh_agg.at[pl.ds(tb, N - tb)])
                pltpu.sync_copy(zdeg_v.at[pl.ds(0, N - tb)],
                                sh_deg.at[pl.ds(tb, N - tb)])
            plsc.subcore_barrier()

            def superload(g):
                e0 = ebase + g * SUPER_E
                pltpu.sync_copy(s_hbm.at[pl.ds(e0, SUPER_E)], srcbig)
                pltpu.sync_copy(d_hbm.at[pl.ds(e0, SUPER_E)], dstbig)

            def prep(j, ib, db):
                off_ = (j % SUPER) * C
                for r in range(C // L):
                    sl = pl.ds(off_ + r * L, L)
                    ib[pl.ds(r * L, L)] = srcbig[sl] + coff
                    db[pl.ds(r * L, L)] = dstbig[sl]

            def gather_start(ib, rb, gs):
                pltpu.async_copy(x_hbm.at[ib], rb, gs)

            def gather_wait(ib, rb, gs):
                pltpu.make_async_copy(x_hbm.at[ib], rb, gs).wait()

            def scatter_start(rb, db, ss, ds_):
                pltpu.async_copy(rb, sh_agg.at[db], ss, add=True)

                @pl.when(c == rel)
                def _():
                    pltpu.async_copy(ones_v, sh_deg.at[db], ds_, add=True)

            def scatter_wait(rb, db, ss, ds_):
                pltpu.make_async_copy(rb, sh_agg.at[db], ss).wait()

                @pl.when(c == rel)
                def _():
                    pltpu.make_async_copy(ones_v, sh_deg.at[db], ds_).wait()

            def halfstep(j, P, PN):
                ib, db, rb, gs, ss, ds_ = P
                ibn, dbn, rbn, gsn, ssn, dsn = PN

                @pl.when(j >= 1)
                def _():
                    scatter_wait(rbn, dbn, ssn, dsn)

                @pl.when((j + 1) % SUPER == 0)
                def _():
                    superload((j + 1) // SUPER)

                prep(j + 1, ibn, dbn)
                gather_wait(ib, rb, gs)
                gather_start(ibn, rbn, gsn)
                scatter_start(rb, db, ss, ds_)

            superload(0)
            prep(0, idx0, dst0)
            gather_start(idx0, rows0, gsem0)

            @pl.loop(0, (CPT - 1) // 2)
            def _(t):
                halfstep(2 * t, B0, B1)
                halfstep(2 * t + 1, B1, B0)

            scatter_wait(rows1, dst1, ssem1, dsem1)
            gather_wait(idx0, rows0, gsem0)
            scatter_start(rows0, dst0, ssem0, dsem0)
            scatter_wait(rows0, dst0, ssem0, dsem0)

            plsc.subcore_barrier()

            off = 0
            for ln in zlens:
                pltpu.sync_copy(sh_agg.at[pl.ds(fb + off, ln)],
                                rows0.at[pl.ds(0, ln)])
                pltpu.sync_copy(rows0.at[pl.ds(0, ln)],
                                agg_hbm.at[pl.ds(coff + fb + off, ln)])
                off += ln

            @pl.when(s == NSUB - 1)
            def _():
                tb = NSUB * FLUSH_ROWS
                pltpu.sync_copy(sh_agg.at[pl.ds(tb, N - tb)],
                                rows0.at[pl.ds(0, N - tb)])
                pltpu.sync_copy(rows0.at[pl.ds(0, N - tb)],
                                agg_hbm.at[pl.ds(coff + tb, N - tb)])

            @pl.when(c == rel)
            def _():
                pltpu.sync_copy(sh_deg.at[pl.ds(fb, FLUSH_ROWS)],
                                dbuf.at[pl.ds(0, FLUSH_ROWS)])
                pltpu.sync_copy(dbuf.at[pl.ds(0, FLUSH_ROWS)],
                                deg_hbm.at[pl.ds(fb, FLUSH_ROWS)])

                @pl.when(s == NSUB - 1)
                def _():
                    tb = NSUB * FLUSH_ROWS
                    pltpu.sync_copy(sh_deg.at[pl.ds(tb, N - tb)],
                                    dbuf.at[pl.ds(0, N - tb)])
                    pltpu.sync_copy(dbuf.at[pl.ds(0, N - tb)],
                                    deg_hbm.at[pl.ds(tb, N - tb)])

            plsc.subcore_barrier()

    return k(xu_cat, xi_cat, src_ui, dst_ui, src_iu, dst_iu)


def _norm_body(xu, aul, auh, du, xi, ail, aih, di,
               pu, pi, gu, bu, gi, bi, ou, oi):
    for x, alo, ahi, d, p, g, b, o in (
            (xu, aul, auh, du, pu, gu, bu, ou),
            (xi, ail, aih, di, pi, gi, bi, oi)):
        a = jnp.concatenate([alo[...], ahi[...]], axis=1)
        h = x[...] + ALPHA * (a * p[...]) / jnp.maximum(d[...], 1.0)
        mu = jnp.mean(h, axis=-1, keepdims=True)
        var = jnp.mean((h - mu) ** 2, axis=-1, keepdims=True)
        o[...] = (h - mu) * lax.rsqrt(var + EPS) * g[...] + b[...]


def _tc_normalize(x_user, aggu_cat, deg_user, x_item, aggi_cat, deg_item,
                  p_iu, p_ui, g_u, b_u, g_i, b_i):
    R = 1000
    bs_feat = pl.BlockSpec((R, D), lambda i: (i, 0))
    bs_lo = pl.BlockSpec((R, DH), lambda i: (i, 0))
    bs_hi = pl.BlockSpec((R, DH), lambda i: (N // R + i, 0))
    bs_deg = pl.BlockSpec((R, 1), lambda i: (i, 0))
    bs_vec = pl.BlockSpec((1, D), lambda i: (0, 0))
    return pl.pallas_call(
        _norm_body,
        grid=(N // R,),
        in_specs=[bs_feat, bs_lo, bs_hi, bs_deg,
                  bs_feat, bs_lo, bs_hi, bs_deg,
                  bs_vec, bs_vec, bs_vec, bs_vec, bs_vec, bs_vec],
        out_specs=[bs_feat, bs_feat],
        out_shape=[jax.ShapeDtypeStruct((N, D), jnp.float32),
                   jax.ShapeDtypeStruct((N, D), jnp.float32)],
    )(x_user, aggu_cat, aggu_cat, deg_user,
      x_item, aggi_cat, aggi_cat, deg_item,
      p_iu.reshape(1, D), p_ui.reshape(1, D),
      g_u.reshape(1, D), b_u.reshape(1, D),
      g_i.reshape(1, D), b_i.reshape(1, D))


def kernel(x_user, x_item, edge_user_item, edge_item_user,
           p_user_item, p_item_user, g_user, b_user, g_item, b_item):
    xu_cat = jnp.concatenate([x_user[:, :DH], x_user[:, DH:]], axis=0)
    xi_cat = jnp.concatenate([x_item[:, :DH], x_item[:, DH:]], axis=0)
    aggi_cat, aggu_cat, deg_item, deg_user = _sc_aggregate(
        xu_cat, xi_cat,
        edge_user_item[0], edge_user_item[1],
        edge_item_user[0], edge_item_user[1])
    out_user, out_item = _tc_normalize(
        x_user, aggu_cat, deg_user.reshape(N, 1),
        x_item, aggi_cat, deg_item.reshape(N, 1),
        p_item_user, p_user_item, g_user, b_user, g_item, b_item)
    return (out_user, out_item)


# speedup vs baseline: 5.8990x; 1.0180x over previous
import functools

import jax
import jax.numpy as jnp
from jax import lax
from jax.experimental import pallas as pl
from jax.experimental.pallas import tpu as pltpu
from jax.experimental.pallas import tpu_sc as plsc

N = 10000
D = 256
E = 160000
DH = 128
C = 80
L = 16
NSUB = 16
EPT = E // NSUB
CPT = EPT // C
DEG_SPLIT = (CPT + 1) // 2
SUPER = 25
SUPER_E = SUPER * C
FLUSH_ROWS = (N // NSUB) // 8 * 8
ALPHA = 0.5
EPS = 1e-5


def _sc_aggregate(x_cat, src, dst):
    mesh = plsc.VectorSubcoreMesh(core_axis_name="core",
                                  subcore_axis_name="subcore")
    out_type = [
        jax.ShapeDtypeStruct((2 * N, DH), jnp.float32),
        jax.ShapeDtypeStruct((2 * N,), jnp.float32),
    ]
    scratch = [
        pltpu.VMEM_SHARED((N, DH), jnp.float32),
        pltpu.VMEM_SHARED((N,), jnp.float32),
        pltpu.VMEM((SUPER_E,), jnp.int32),
        pltpu.VMEM((SUPER_E,), jnp.int32),
        pltpu.VMEM((C,), jnp.int32),
        pltpu.VMEM((C,), jnp.int32),
        pltpu.VMEM((C, DH), jnp.float32),
        pltpu.VMEM((C,), jnp.int32),
        pltpu.VMEM((C,), jnp.int32),
        pltpu.VMEM((C, DH), jnp.float32),
        pltpu.VMEM((C,), jnp.float32),
        pltpu.VMEM((C,), jnp.float32),
        pltpu.VMEM((FLUSH_ROWS + 16,), jnp.float32),
        pltpu.SemaphoreType.DMA,
        pltpu.SemaphoreType.DMA,
        pltpu.SemaphoreType.DMA,
        pltpu.SemaphoreType.DMA,
        pltpu.SemaphoreType.DMA,
        pltpu.SemaphoreType.DMA,
    ]

    @functools.partial(pl.kernel, mesh=mesh, out_type=out_type,
                       scratch_types=scratch)
    def k(x_hbm, s_hbm, d_hbm, agg_hbm, deg_hbm,
          sh_agg, sh_deg, srcbig, dstbig,
          idx0, dst0, rows0, idx1, dst1, rows1,
          ones_v, zdeg_v, dbuf,
          gsem0, gsem1, ssem0, ssem1, dsem0, dsem1):
        c = lax.axis_index("core")
        s = lax.axis_index("subcore")
        coff = c * N
        ebase = s * EPT

        for j in range(C // L):
            ones_v[pl.ds(j * L, L)] = jnp.full((L,), 1.0, jnp.float32)
            zdeg_v[pl.ds(j * L, L)] = jnp.zeros((L,), jnp.float32)

        @pl.loop(0, C)
        def _(i):
            for j in range(DH // L):
                rows0[i, pl.ds(j * L, L)] = jnp.zeros((L,), jnp.float32)

        fb = s * FLUSH_ROWS
        zlens = [C] * (FLUSH_ROWS // C) + [FLUSH_ROWS % C]
        off = 0
        for ln in zlens:
            pltpu.sync_copy(rows0.at[pl.ds(0, ln)],
                            sh_agg.at[pl.ds(fb + off, ln)])
            pltpu.sync_copy(zdeg_v.at[pl.ds(0, ln)],
                            sh_deg.at[pl.ds(fb + off, ln)])
            off += ln

        @pl.when(s == NSUB - 1)
        def _():
            tb = NSUB * FLUSH_ROWS
            pltpu.sync_copy(rows0.at[pl.ds(0, N - tb)],
                            sh_agg.at[pl.ds(tb, N - tb)])
            pltpu.sync_copy(zdeg_v.at[pl.ds(0, N - tb)],
                            sh_deg.at[pl.ds(tb, N - tb)])
        plsc.subcore_barrier()

        def deg_on(j):
            return ((j < DEG_SPLIT) & (c == 0)) | ((j >= DEG_SPLIT) & (c == 1))

        def superload(g):
            e0 = ebase + g * SUPER_E
            pltpu.sync_copy(s_hbm.at[pl.ds(e0, SUPER_E)], srcbig)
            pltpu.sync_copy(d_hbm.at[pl.ds(e0, SUPER_E)], dstbig)

        def prep(j, ib, db):
            off_ = (j % SUPER) * C
            for r in range(C // L):
                sl = pl.ds(off_ + r * L, L)
                ib[pl.ds(r * L, L)] = srcbig[sl] + coff
                db[pl.ds(r * L, L)] = dstbig[sl]

        def scatter_start(j, rb, db, ss, ds_):
            pltpu.async_copy(rb, sh_agg.at[db], ss, add=True)

            @pl.when(deg_on(j))
            def _():
                pltpu.async_copy(ones_v, sh_deg.at[db], ds_, add=True)

        def scatter_wait(j, rb, db, ss, ds_):
            pltpu.make_async_copy(rb, sh_agg.at[db], ss).wait()

            @pl.when(deg_on(j))
            def _():
                pltpu.make_async_copy(ones_v, sh_deg.at[db], ds_).wait()

        def halfstep(j, P, PN):
            ib, db, rb, gs, ss, ds_ = P
            ibn, dbn, rbn, gsn, ssn, dsn = PN

            @pl.when(j >= 1)
            def _():
                scatter_wait(j - 1, rbn, dbn, ssn, dsn)

            @pl.when((j + 1) % SUPER == 0)
            def _():
                superload((j + 1) // SUPER)

            prep(j + 1, ibn, dbn)
            pltpu.make_async_copy(x_hbm.at[ib], rb, gs).wait()
            pltpu.async_copy(x_hbm.at[ibn], rbn, gsn)
            scatter_start(j, rb, db, ss, ds_)

        B0 = (idx0, dst0, rows0, gsem0, ssem0, dsem0)
        B1 = (idx1, dst1, rows1, gsem1, ssem1, dsem1)

        superload(0)
        prep(0, idx0, dst0)
        pltpu.async_copy(x_hbm.at[idx0], rows0, gsem0)

        @pl.loop(0, (CPT - 1) // 2)
        def _(t):
            halfstep(2 * t, B0, B1)
            halfstep(2 * t + 1, B1, B0)

        scatter_wait(CPT - 2, rows1, dst1, ssem1, dsem1)
        pltpu.make_async_copy(x_hbm.at[idx0], rows0, gsem0).wait()
        scatter_start(CPT - 1, rows0, dst0, ssem0, dsem0)
        scatter_wait(CPT - 1, rows0, dst0, ssem0, dsem0)

        plsc.subcore_barrier()

        off = 0
        for ln in zlens:
            pltpu.sync_copy(sh_agg.at[pl.ds(fb + off, ln)],
                            rows0.at[pl.ds(0, ln)])
            pltpu.sync_copy(rows0.at[pl.ds(0, ln)],
                            agg_hbm.at[pl.ds(coff + fb + off, ln)])
            off += ln
        pltpu.sync_copy(sh_deg.at[pl.ds(fb, FLUSH_ROWS)],
                        dbuf.at[pl.ds(0, FLUSH_ROWS)])
        pltpu.sync_copy(dbuf.at[pl.ds(0, FLUSH_ROWS)],
                        deg_hbm.at[pl.ds(coff + fb, FLUSH_ROWS)])

        @pl.when(s == NSUB - 1)
        def _():
            tb = NSUB * FLUSH_ROWS
            pltpu.sync_copy(sh_agg.at[pl.ds(tb, N - tb)],
                            rows0.at[pl.ds(0, N - tb)])
            pltpu.sync_copy(rows0.at[pl.ds(0, N - tb)],
                            agg_hbm.at[pl.ds(coff + tb, N - tb)])
            pltpu.sync_copy(sh_deg.at[pl.ds(tb, N - tb)],
                            dbuf.at[pl.ds(0, N - tb)])
            pltpu.sync_copy(dbuf.at[pl.ds(0, N - tb)],
                            deg_hbm.at[pl.ds(coff + tb, N - tb)])

    return k(x_cat, src, dst)


def _norm_body(xu, aul, auh, dul, duh, xi, ail, aih, dil, dih,
               pu, pi, gu, bu, gi, bi, ou, oi):
    for x, alo, ahi, dlo, dhi, p, g, b, o in (
            (xu, aul, auh, dul, duh, pu, gu, bu, ou),
            (xi, ail, aih, dil, dih, pi, gi, bi, oi)):
        a = jnp.concatenate([alo[...], ahi[...]], axis=1)
        d = dlo[...] + dhi[...]
        h = x[...] + ALPHA * (a * p[...]) / jnp.maximum(d, 1.0)
        mu = jnp.mean(h, axis=-1, keepdims=True)
        var = jnp.mean((h - mu) ** 2, axis=-1, keepdims=True)
        o[...] = (h - mu) * lax.rsqrt(var + EPS) * g[...] + b[...]


def _tc_normalize(x_user, aggu_cat, degu_cat, x_item, aggi_cat, degi_cat,
                  p_iu, p_ui, g_u, b_u, g_i, b_i):
    R = 1000
    bs_feat = pl.BlockSpec((R, D), lambda i: (i, 0))
    bs_lo = pl.BlockSpec((R, DH), lambda i: (i, 0))
    bs_hi = pl.BlockSpec((R, DH), lambda i: (N // R + i, 0))
    bs_dlo = pl.BlockSpec((R, 1), lambda i: (i, 0))
    bs_dhi = pl.BlockSpec((R, 1), lambda i: (N // R + i, 0))
    bs_vec = pl.BlockSpec((1, D), lambda i: (0, 0))
    return pl.pallas_call(
        _norm_body,
        grid=(N // R,),
        in_specs=[bs_feat, bs_lo, bs_hi, bs_dlo, bs_dhi,
                  bs_feat, bs_lo, bs_hi, bs_dlo, bs_dhi,
                  bs_vec, bs_vec, bs_vec, bs_vec, bs_vec, bs_vec],
        out_specs=[bs_feat, bs_feat],
        out_shape=[jax.ShapeDtypeStruct((N, D), jnp.float32),
                   jax.ShapeDtypeStruct((N, D), jnp.float32)],
    )(x_user, aggu_cat, aggu_cat, degu_cat, degu_cat,
      x_item, aggi_cat, aggi_cat, degi_cat, degi_cat,
      p_iu.reshape(1, D), p_ui.reshape(1, D),
      g_u.reshape(1, D), b_u.reshape(1, D),
      g_i.reshape(1, D), b_i.reshape(1, D))


def kernel(x_user, x_item, edge_user_item, edge_item_user,
           p_user_item, p_item_user, g_user, b_user, g_item, b_item):
    xu_cat = jnp.concatenate([x_user[:, :DH], x_user[:, DH:]], axis=0)
    xi_cat = jnp.concatenate([x_item[:, :DH], x_item[:, DH:]], axis=0)
    aggi_cat, degi_cat = _sc_aggregate(
        xu_cat, edge_user_item[0], edge_user_item[1])
    aggu_cat, degu_cat = _sc_aggregate(
        xi_cat, edge_item_user[0], edge_item_user[1])
    out_user, out_item = _tc_normalize(
        x_user, aggu_cat, degu_cat.reshape(2 * N, 1),
        x_item, aggi_cat, degi_cat.reshape(2 * N, 1),
        p_item_user, p_user_item, g_user, b_user, g_item, b_item)
    return (out_user, out_item)
